# baseline (speedup 1.0000x reference)
import functools

import jax
import jax.numpy as jnp
from jax import lax
from jax.experimental import pallas as pl
from jax.experimental.pallas import tpu as pltpu
from jax.experimental.pallas import tpu_sc as plsc

VOCAB = 100000
DIM = 128
BATCH = 16384
HIST = 50
TOT = BATCH * HIST

_info = plsc.get_sparse_core_info()
NC, NS = _info.num_cores, _info.num_subcores
NW = NC * NS
BAT_W = BATCH // NW
PER_W = TOT // NW
HIST_PAD = 56
IDX_W = BAT_W * HIST_PAD
NCH = BAT_W
NB = 8
DEPTH = 4
NSTEP = NCH // NB


def _make_gather():
    mesh = plsc.VectorSubcoreMesh(core_axis_name="c", subcore_axis_name="s")

    @functools.partial(
        pl.kernel,
        mesh=mesh,
        out_type=jax.ShapeDtypeStruct((BATCH, HIST, DIM), jnp.float32),
        scratch_types=(
            [pltpu.VMEM((IDX_W,), jnp.int32)]
            + [pltpu.VMEM((HIST, DIM), jnp.float32) for _ in range(NB)]
            + [pltpu.SemaphoreType.DMA for _ in range(2 * NB)]
        ),
    )
    def gather_kernel(idx_hbm, table_hbm, out_hbm, idx_v, *bufs_and_sems):
        rows = bufs_and_sems[:NB]
        gsem = bufs_and_sems[NB : 2 * NB]
        wsem = bufs_and_sems[2 * NB : 3 * NB]

        wid = lax.axis_index("s") * NC + lax.axis_index("c")
        bbase = wid * BAT_W

        pltpu.sync_copy(idx_hbm.at[pl.ds(wid * IDX_W, IDX_W)], idx_v)

        def start_gather(t, b):
            idx_slice = idx_v.at[pl.ds(t * HIST_PAD, HIST)]
            pltpu.async_copy(table_hbm.at[idx_slice], rows[b], gsem[b])

        def start_store(t, b):
            pltpu.async_copy(rows[b], out_hbm.at[bbase + t], wsem[b])

        def wait_store(b):
            pltpu.make_async_copy(rows[b], out_hbm.at[bbase], wsem[b]).wait()

        def wait_gather(b):
            pltpu.make_async_copy(
                table_hbm.at[idx_v.at[pl.ds(0, HIST)]], rows[b], gsem[b]
            ).wait()

        for d in range(DEPTH):
            start_gather(d, d)

        def step_body(s, carry):
            for b in range(NB):
                t = s * NB + b
                gn = t + DEPTH
                bg = (b + DEPTH) % NB

                @pl.when(jnp.logical_and(gn >= NB, gn < NCH))
                def _():
                    wait_store(bg)

                @pl.when(gn < NCH)
                def _():
                    start_gather(gn, bg)

                wait_gather(b)
                start_store(t, b)
            return carry

        lax.fori_loop(0, NSTEP, step_body, 0)

        for b in range(NB):
            wait_store(b)

    return gather_kernel


_gather = _make_gather()


def kernel(indices, table):
    idx = indices.astype(jnp.int32)
    idx = jnp.pad(idx, ((0, 0), (0, HIST_PAD - HIST)))
    flat = jnp.reshape(idx, (BATCH * HIST_PAD,))
    return _gather(flat, table)
